# baseline (speedup 1.0000x reference)
import functools

import jax
import jax.numpy as jnp
from jax import lax
from jax.experimental import pallas as pl
from jax.experimental.pallas import tpu as pltpu
from jax.experimental.pallas import tpu_sc as plsc

VOCAB = 100000
HB = 256
HT = 256
SEQ = 50
NS = 16
LANES = 16
CH_MAIN = 6256
CH_LAST = VOCAB - (NS - 1) * CH_MAIN
NIT_MAIN = CH_MAIN // LANES
NIT_LAST = CH_LAST // LANES


def _body(x_hbm, uni_hbm, bi_hbm, tri_hbm, al_hbm, out_hbm,
          x_v, a_v, u_v, b_v, t_v, o_v, loc_v, all_v, shared):
    cid = lax.axis_index("c")
    wid = lax.axis_index("s")
    is_last = wid == NS - 1
    base = wid * CH_MAIN

    pltpu.sync_copy(x_hbm, x_v.at[pl.ds(0, SEQ)])
    pltpu.sync_copy(al_hbm, a_v.at[pl.ds(0, 3)])

    vt = x_v[pl.ds(48, 16)]
    t0 = vt[0]
    t1 = vt[1]
    bi_idx = jnp.bitwise_and(t1, HB - 1)
    tri_idx = jnp.bitwise_and(t0 * 31 + t1, HT - 1)

    b_off = bi_idx * VOCAB + base
    t_off = tri_idx * VOCAB + base

    @pl.when(jnp.logical_not(is_last))
    def _():
        pltpu.sync_copy(uni_hbm.at[pl.ds(base, CH_MAIN)], u_v.at[pl.ds(0, CH_MAIN)])
        pltpu.sync_copy(bi_hbm.at[pl.ds(b_off, CH_MAIN)], b_v.at[pl.ds(0, CH_MAIN)])
        pltpu.sync_copy(tri_hbm.at[pl.ds(t_off, CH_MAIN)], t_v.at[pl.ds(0, CH_MAIN)])

    @pl.when(is_last)
    def _():
        pltpu.sync_copy(uni_hbm.at[pl.ds(base, CH_LAST)], u_v.at[pl.ds(0, CH_LAST)])
        pltpu.sync_copy(bi_hbm.at[pl.ds(b_off, CH_LAST)], b_v.at[pl.ds(0, CH_LAST)])
        pltpu.sync_copy(tri_hbm.at[pl.ds(t_off, CH_LAST)], t_v.at[pl.ds(0, CH_LAST)])

    nmy = jnp.where(is_last, NIT_LAST, NIT_MAIN)
    zero = jnp.zeros((16,), jnp.float32)

    def sbody(i, carry):
        au, ab, at_ = carry
        off = i * LANES
        ok = i < nmy
        u16 = jnp.where(ok, u_v[pl.ds(off, 16)], 0.0)
        b16 = jnp.where(ok, b_v[pl.ds(off, 16)], 0.0)
        t16 = jnp.where(ok, t_v[pl.ds(off, 16)], 0.0)
        return au + u16, ab + b16, at_ + t16

    au, ab, at_ = lax.fori_loop(0, NIT_MAIN, sbody, (zero, zero, zero))

    loc_v[pl.ds(0, 16)] = au
    loc_v[pl.ds(16, 16)] = ab
    loc_v[pl.ds(32, 16)] = at_
    pltpu.sync_copy(loc_v, shared.at[pl.ds(wid * 48, 48)])
    plsc.subcore_barrier()
    pltpu.sync_copy(shared, all_v)

    su = zero
    sb = zero
    st = zero
    for w in range(NS):
        su = su + all_v[pl.ds(w * 48, 16)]
        sb = sb + all_v[pl.ds(w * 48 + 16, 16)]
        st = st + all_v[pl.ds(w * 48 + 32, 16)]
    s_uni = su[0]
    s_bi = sb[0]
    s_tri = st[0]
    for i in range(1, 16):
        s_uni = s_uni + su[i]
        s_bi = s_bi + sb[i]
        s_tri = s_tri + st[i]

    va = a_v[...]
    cu = jnp.broadcast_to(va[0], (16,)) / jnp.broadcast_to(s_uni, (16,))
    cb = jnp.broadcast_to(va[1], (16,)) / jnp.broadcast_to(s_bi, (16,))
    ct = jnp.broadcast_to(va[2], (16,)) / jnp.broadcast_to(s_tri, (16,))

    mine = (wid < 8) == (cid == 0)

    @pl.when(mine)
    def _():
        def obody(i, carry):
            off = i * LANES
            o_v[pl.ds(off, 16)] = (u_v[pl.ds(off, 16)] * cu
                                   + b_v[pl.ds(off, 16)] * cb
                                   + t_v[pl.ds(off, 16)] * ct)
            return carry

        lax.fori_loop(0, NIT_MAIN, obody, 0)

    @pl.when(mine & jnp.logical_not(is_last))
    def _():
        pltpu.sync_copy(o_v.at[pl.ds(0, CH_MAIN)], out_hbm.at[pl.ds(base, CH_MAIN)])

    @pl.when(mine & is_last)
    def _():
        pltpu.sync_copy(o_v.at[pl.ds(0, CH_LAST)], out_hbm.at[pl.ds(base, CH_LAST)])


@functools.partial(jax.jit, static_argnames=())
def kernel(x, uni_counts, bi_counts, tri_counts, alphas):
    run = pl.kernel(
        _body,
        out_type=jax.ShapeDtypeStruct((VOCAB,), jnp.float32),
        mesh=plsc.VectorSubcoreMesh(core_axis_name="c", subcore_axis_name="s"),
        scratch_types=[
            pltpu.VMEM((64,), jnp.int32),
            pltpu.VMEM((16,), jnp.float32),
            pltpu.VMEM((CH_MAIN,), jnp.float32),
            pltpu.VMEM((CH_MAIN,), jnp.float32),
            pltpu.VMEM((CH_MAIN,), jnp.float32),
            pltpu.VMEM((CH_MAIN,), jnp.float32),
            pltpu.VMEM((48,), jnp.float32),
            pltpu.VMEM((NS * 48,), jnp.float32),
            pltpu.VMEM_SHARED((NS * 48,), jnp.float32),
        ],
    )
    return run(
        x.astype(jnp.int32),
        uni_counts,
        bi_counts.reshape(HB * VOCAB),
        tri_counts.reshape(HT * VOCAB),
        alphas,
    )


# speedup vs baseline: 2.0213x; 2.0213x over previous
import functools

import jax
import jax.numpy as jnp
from jax import lax
from jax.experimental import pallas as pl
from jax.experimental.pallas import tpu as pltpu
from jax.experimental.pallas import tpu_sc as plsc

VOCAB = 100000
HB = 256
HT = 256
SEQ = 50
NS = 16
LANES = 16
CW = 6272
NITW = CW // LANES
LAST_BASE = 93696
OVER_GROUPS = (NS - 1) * CW // LANES - LAST_BASE // LANES
TAIL_OFF = 99968
TAIL = VOCAB - TAIL_OFF


def _body(x_hbm, uni_hbm, bi_hbm, tri_hbm, al_hbm, out_hbm,
          x_v, a_v, u_v, sb_v, st_v, o_v, tu_v, tb_v, tt_v, to_v,
          loc_v, all_v, shared):
    cid = lax.axis_index("c")
    wid = lax.axis_index("s")
    is_last = wid == NS - 1
    base = pl.multiple_of(jnp.where(is_last, LAST_BASE, wid * CW), 128)

    pltpu.sync_copy(x_hbm, x_v.at[pl.ds(0, SEQ)])
    pltpu.sync_copy(al_hbm, a_v.at[pl.ds(0, 3)])

    vt = x_v[pl.ds(48, 16)]
    t0 = vt[0]
    t1 = vt[1]
    bi_idx = jnp.bitwise_and(t1, HB - 1)
    tri_idx = jnp.bitwise_and(t0 * 31 + t1, HT - 1)
    rb_b = pl.multiple_of(jnp.bitwise_and(bi_idx, ~7), 8)
    rb_t = pl.multiple_of(jnp.bitwise_and(tri_idx, ~7), 8)
    sub_b = jnp.bitwise_and(bi_idx, 7)
    sub_t = jnp.bitwise_and(tri_idx, 7)

    pltpu.sync_copy(uni_hbm.at[pl.ds(base, CW)], u_v)
    pltpu.sync_copy(bi_hbm.at[pl.ds(rb_b, 8), pl.ds(base, CW)], sb_v)
    pltpu.sync_copy(tri_hbm.at[pl.ds(rb_t, 8), pl.ds(base, CW)], st_v)

    @pl.when(is_last)
    def _():
        pltpu.sync_copy(uni_hbm.at[pl.ds(TAIL_OFF, TAIL)], tu_v)
        pltpu.sync_copy(bi_hbm.at[pl.ds(rb_b, 8), pl.ds(TAIL_OFF, TAIL)], tb_v)
        pltpu.sync_copy(tri_hbm.at[pl.ds(rb_t, 8), pl.ds(TAIL_OFF, TAIL)], tt_v)

    i_lo = jnp.where(is_last, OVER_GROUPS, 0)
    zero = jnp.zeros((16,), jnp.float32)

    def sbody(i, carry):
        au, ab, at_ = carry
        off = i * LANES
        ok = i >= i_lo
        u16 = jnp.where(ok, u_v[pl.ds(off, 16)], 0.0)
        b16 = jnp.where(ok, sb_v[sub_b, pl.ds(off, 16)], 0.0)
        t16 = jnp.where(ok, st_v[sub_t, pl.ds(off, 16)], 0.0)
        return au + u16, ab + b16, at_ + t16

    au, ab, at_ = lax.fori_loop(0, NITW, sbody, (zero, zero, zero))

    @pl.when(is_last)
    def _():
        au2, ab2, at2 = au, ab, at_
        for g in range(TAIL // LANES):
            off = g * LANES
            au2 = au2 + tu_v[pl.ds(off, 16)]
            ab2 = ab2 + tb_v[sub_b, pl.ds(off, 16)]
            at2 = at2 + tt_v[sub_t, pl.ds(off, 16)]
        loc_v[pl.ds(0, 16)] = au2
        loc_v[pl.ds(16, 16)] = ab2
        loc_v[pl.ds(32, 16)] = at2

    @pl.when(jnp.logical_not(is_last))
    def _():
        loc_v[pl.ds(0, 16)] = au
        loc_v[pl.ds(16, 16)] = ab
        loc_v[pl.ds(32, 16)] = at_

    pltpu.sync_copy(loc_v, shared.at[pl.ds(wid * 48, 48)])
    plsc.subcore_barrier()
    pltpu.sync_copy(shared, all_v)

    su = zero
    sb = zero
    st = zero
    for w in range(NS):
        su = su + all_v[pl.ds(w * 48, 16)]
        sb = sb + all_v[pl.ds(w * 48 + 16, 16)]
        st = st + all_v[pl.ds(w * 48 + 32, 16)]
    s_uni = su[0]
    s_bi = sb[0]
    s_tri = st[0]
    for i in range(1, 16):
        s_uni = s_uni + su[i]
        s_bi = s_bi + sb[i]
        s_tri = s_tri + st[i]

    va = a_v[...]
    cu = jnp.broadcast_to(va[0], (16,)) / jnp.broadcast_to(s_uni, (16,))
    cb = jnp.broadcast_to(va[1], (16,)) / jnp.broadcast_to(s_bi, (16,))
    ct = jnp.broadcast_to(va[2], (16,)) / jnp.broadcast_to(s_tri, (16,))

    mine = (wid < 8) == (cid == 0)

    @pl.when(mine)
    def _():
        def obody(i, carry):
            off = i * LANES
            o_v[pl.ds(off, 16)] = (u_v[pl.ds(off, 16)] * cu
                                   + sb_v[sub_b, pl.ds(off, 16)] * cb
                                   + st_v[sub_t, pl.ds(off, 16)] * ct)
            return carry

        lax.fori_loop(0, NITW, obody, 0)

    @pl.when(mine & jnp.logical_not(is_last))
    def _():
        pltpu.sync_copy(o_v, out_hbm.at[pl.ds(base, CW)])

    @pl.when(mine & is_last)
    def _():
        for g in range(TAIL // LANES):
            off = g * LANES
            to_v[pl.ds(off, 16)] = (tu_v[pl.ds(off, 16)] * cu
                                    + tb_v[sub_b, pl.ds(off, 16)] * cb
                                    + tt_v[sub_t, pl.ds(off, 16)] * ct)
        own = OVER_GROUPS * LANES
        pltpu.sync_copy(o_v.at[pl.ds(own, CW - own)],
                        out_hbm.at[pl.ds(LAST_BASE + own, CW - own)])
        pltpu.sync_copy(to_v, out_hbm.at[pl.ds(TAIL_OFF, TAIL)])


@functools.partial(jax.jit, static_argnames=())
def kernel(x, uni_counts, bi_counts, tri_counts, alphas):
    run = pl.kernel(
        _body,
        out_type=jax.ShapeDtypeStruct((VOCAB,), jnp.float32),
        mesh=plsc.VectorSubcoreMesh(core_axis_name="c", subcore_axis_name="s"),
        scratch_types=[
            pltpu.VMEM((64,), jnp.int32),
            pltpu.VMEM((16,), jnp.float32),
            pltpu.VMEM((CW,), jnp.float32),
            pltpu.VMEM((8, CW), jnp.float32),
            pltpu.VMEM((8, CW), jnp.float32),
            pltpu.VMEM((CW,), jnp.float32),
            pltpu.VMEM((TAIL,), jnp.float32),
            pltpu.VMEM((8, TAIL), jnp.float32),
            pltpu.VMEM((8, TAIL), jnp.float32),
            pltpu.VMEM((TAIL,), jnp.float32),
            pltpu.VMEM((48,), jnp.float32),
            pltpu.VMEM((NS * 48,), jnp.float32),
            pltpu.VMEM_SHARED((NS * 48,), jnp.float32),
        ],
    )
    return run(
        x.astype(jnp.int32),
        uni_counts,
        bi_counts,
        tri_counts,
        alphas,
    )
